# Initial kernel scaffold; baseline (speedup 1.0000x reference)
import jax
import jax.numpy as jnp
from jax.experimental import pallas as pl


def kernel(input):
    raise NotImplementedError("write your pallas kernel here")


# speedup vs baseline: 64.4133x; 64.4133x over previous
import functools

import jax
import jax.numpy as jnp
from jax.experimental import pallas as pl
from jax.experimental.pallas import tpu as pltpu


def _topk_mask_kernel(x_ref, o_ref, *, k):
    x = x_ref[...]
    n = x.shape[-1]
    u = jax.lax.bitcast_convert_type(x, jnp.uint32)
    sign = jnp.uint32(0x80000000)
    ku = jnp.where(u >= sign, ~u, u | sign)

    def body(t, prefix):
        bit = jnp.left_shift(jnp.uint32(1), jnp.uint32(31) - t.astype(jnp.uint32))
        tryv = prefix | bit
        c = jnp.sum((ku >= tryv).astype(jnp.int32), axis=1, keepdims=True)
        return jnp.where(c >= k, tryv, prefix)

    T = jax.lax.fori_loop(0, 32, body, jnp.zeros((x.shape[0], 1), jnp.uint32))

    gt = ku > T
    eq = ku == T
    count_gt = jnp.sum(gt.astype(jnp.int32), axis=1, keepdims=True)
    need = k - count_gt

    csum = eq.astype(jnp.int32)
    d = 1
    while d < n:
        shifted = jnp.concatenate(
            [jnp.zeros((x.shape[0], d), jnp.int32), csum[:, : n - d]], axis=1
        )
        csum = csum + shifted
        d *= 2

    mask = gt | (eq & (csum <= need))
    o_ref[...] = x * mask.astype(x.dtype)


@jax.jit
def kernel(input):
    b, n = input.shape
    k = n // 2
    return pl.pallas_call(
        functools.partial(_topk_mask_kernel, k=k),
        out_shape=jax.ShapeDtypeStruct((b, n), input.dtype),
    )(input)
